# baseline (speedup 1.0000x reference)
import functools

import jax
import jax.numpy as jnp
from jax import lax
from jax.experimental import pallas as pl
from jax.experimental.pallas import tpu as pltpu
from jax.experimental.pallas import tpu_sc as plsc

V = 50000
H = 256
B = 16
L = 256
S = 4
R = 2
NT = 16

_NC = 2
_NS = 16
_NW = _NC * _NS
_TOTAL = B * L * S
_PER_W = _TOTAL // _NW
_CHUNK = 256

@functools.cache
def _sc_gather_fn():
    mesh = plsc.VectorSubcoreMesh(core_axis_name="c", subcore_axis_name="s",
                                  num_cores=_NC, num_subcores=_NS)

    @functools.partial(
        pl.kernel,
        mesh=mesh,
        out_type=jax.ShapeDtypeStruct((_TOTAL, H), jnp.float32),
        scratch_types=[
            pltpu.VMEM((_PER_W,), jnp.int32),
            pltpu.VMEM((_CHUNK, H), jnp.float32),
            pltpu.SemaphoreType.DMA,
        ],
    )
    def _sc_gather(table_hbm, idx_hbm, out_hbm, idx_v, rows_v, sem):
        wid = lax.axis_index("s") * _NC + lax.axis_index("c")
        base = wid * _PER_W
        pltpu.sync_copy(idx_hbm.at[pl.ds(base, _PER_W)], idx_v)
        for c in range(_PER_W // _CHUNK):
            pltpu.async_copy(
                table_hbm.at[idx_v.at[pl.ds(c * _CHUNK, _CHUNK)]], rows_v, sem
            ).wait()
            pltpu.sync_copy(rows_v,
                            out_hbm.at[pl.ds(base + c * _CHUNK, _CHUNK)])

    return _sc_gather


def _mm(a, b):
    return lax.dot_general(a, b, (((1,), (0,)), ((), ())),
                           preferred_element_type=jnp.float32)


def _mmT(a, b):
    return lax.dot_general(a, b, (((0,), (0,)), ((), ())),
                           preferred_element_type=jnp.float32)


def _gru_step(A, s, W_ref, U_ref, b_ref, mask, transpose):
    sb = s.astype(jnp.bfloat16)
    if transpose:
        m = _mmT(A, sb)
    else:
        m = _mm(A, sb)
    mb = m.astype(jnp.bfloat16)
    z = jax.nn.sigmoid(_mm(mb, W_ref[0]) + _mm(sb, U_ref[0]) + b_ref[0])
    r = jax.nn.sigmoid(_mm(mb, W_ref[1]) + _mm(sb, U_ref[1]) + b_ref[1])
    n = jnp.tanh(_mm(mb, W_ref[2]) + r * _mm(sb, U_ref[2]) + b_ref[2])
    return ((1.0 - z) * s + z * n) * mask


def _tc_body(len_ref, twf_ref, twb_ref, g_ref, tok_ref, deps_ref, et_ref,
             cm_ref, Wf_ref, Uf_ref, bf_ref, Wb_ref, Ub_ref, bb_ref,
             Wo1_ref, Wo2_ref, bo_ref, out_ref, hid_ref):
    b = pl.program_id(0)
    g2 = g_ref[0]
    tok = tok_ref[0]
    e = jnp.zeros((L, H), jnp.float32)
    for s in range(S):
        pad = (tok[:, s:s + 1] != 0).astype(jnp.float32)
        e = e + g2[:, s * H:(s + 1) * H] * pad
    lenb = len_ref[b]
    mask = (lax.broadcasted_iota(jnp.int32, (L, 1), 0) < lenb
            ).astype(jnp.float32)
    sf = e
    sbk = e
    for r in range(R):
        base = deps_ref[0, r] * cm_ref[0, r] * jnp.float32(1.0 / L)
        et = et_ref[0, r]
        accf = jnp.take_along_axis(
            jnp.broadcast_to(twf_ref[...], (L, NT)), et, axis=1)
        accb = jnp.take_along_axis(
            jnp.broadcast_to(twb_ref[...], (L, NT)), et, axis=1)
        Af = (base * accf).astype(jnp.bfloat16)
        Ab = (base * accb).astype(jnp.bfloat16)
        sf = _gru_step(Af, sf, Wf_ref, Uf_ref, bf_ref, mask, False)
        sbk = _gru_step(Ab, sbk, Wb_ref, Ub_ref, bb_ref, mask, True)
    sfb = sf.astype(jnp.bfloat16)
    sbb = sbk.astype(jnp.bfloat16)
    out_ref[0] = _mm(sfb, Wo1_ref[...]) + _mm(sbb, Wo2_ref[...]) + bo_ref[0]
    inv = jnp.float32(1.0) / jnp.maximum(lenb, 1).astype(jnp.float32)
    hf = (sf.sum(axis=0, keepdims=True) * inv).astype(jnp.bfloat16)
    hb = (sbk.sum(axis=0, keepdims=True) * inv).astype(jnp.bfloat16)
    hid_ref[0] = _mm(hf, Wo1_ref[...]) + _mm(hb, Wo2_ref[...]) + bo_ref[0]


def _tc_call(lengths, tw_f, tw_b, g4, tok4, deps, edge_types, cell_mask,
             Wfb, Ufb, bf, Wbb, Ubb, bb, Wo1, Wo2, bo):
    smem = pl.BlockSpec(memory_space=pltpu.SMEM)
    full3 = lambda shape: pl.BlockSpec(shape, lambda b: (0, 0, 0))
    full2 = lambda shape: pl.BlockSpec(shape, lambda b: (0, 0))
    return pl.pallas_call(
        _tc_body,
        grid=(B,),
        in_specs=[
            smem,
            pl.BlockSpec((NT,), lambda b: (0,)),
            pl.BlockSpec((NT,), lambda b: (0,)),
            pl.BlockSpec((1, L, S * H), lambda b: (b, 0, 0)),
            pl.BlockSpec((1, L, S), lambda b: (b, 0, 0)),
            pl.BlockSpec((1, R, L, L), lambda b: (b, 0, 0, 0)),
            pl.BlockSpec((1, R, L, L), lambda b: (b, 0, 0, 0)),
            pl.BlockSpec((1, R, L, L), lambda b: (b, 0, 0, 0)),
            full3((3, H, H)),
            full3((3, H, H)),
            full2((3, H)),
            full3((3, H, H)),
            full3((3, H, H)),
            full2((3, H)),
            full2((H, H)),
            full2((H, H)),
            full2((1, H)),
        ],
        out_specs=[
            pl.BlockSpec((1, L, H), lambda b: (b, 0, 0)),
            pl.BlockSpec((1, 1, H), lambda b: (b, 0, 0)),
        ],
        out_shape=[
            jax.ShapeDtypeStruct((B, L, H), jnp.float32),
            jax.ShapeDtypeStruct((B, 1, H), jnp.float32),
        ],
    )(lengths, tw_f, tw_b, g4, tok4, deps, edge_types, cell_mask,
      Wfb, Ufb, bf, Wbb, Ubb, bb, Wo1, Wo2, bo)


def kernel(input_seqs, input_lengths, deps, edge_types, cell_mask, emb,
           type_w_f, type_w_b, Wf, Uf, bf, Wb, Ub, bb, W_out, b_out):
    flat_idx = input_seqs.reshape(_TOTAL)
    gathered = _sc_gather_fn()(emb, flat_idx).reshape(B, L, S * H)
    bf16 = jnp.bfloat16
    outputs, hidden3 = _tc_call(
        input_lengths, type_w_f, type_w_b, gathered, input_seqs, deps,
        edge_types,
        cell_mask, Wf.astype(bf16), Uf.astype(bf16), bf,
        Wb.astype(bf16), Ub.astype(bf16), bb,
        W_out[:H].astype(bf16), W_out[H:].astype(bf16), b_out.reshape(1, H))
    return outputs, hidden3.reshape(B, H)


# speedup vs baseline: 1.2777x; 1.2777x over previous
import functools

import jax
import jax.numpy as jnp
from jax import lax
from jax.experimental import pallas as pl
from jax.experimental.pallas import tpu as pltpu
from jax.experimental.pallas import tpu_sc as plsc

V = 50000
H = 256
B = 16
L = 256
S = 4
R = 2
NT = 16

_NC = 2
_NS = 16
_NW = _NC * _NS
_TOTAL = B * L * S
_PER_W = _TOTAL // _NW
_CHUNK = 256

@functools.cache
def _sc_gather_fn():
    mesh = plsc.VectorSubcoreMesh(core_axis_name="c", subcore_axis_name="s",
                                  num_cores=_NC, num_subcores=_NS)

    @functools.partial(
        pl.kernel,
        mesh=mesh,
        out_type=jax.ShapeDtypeStruct((_TOTAL, H), jnp.float32),
        scratch_types=[
            pltpu.VMEM((_PER_W,), jnp.int32),
            pltpu.VMEM((_CHUNK, H), jnp.float32),
            pltpu.SemaphoreType.DMA,
        ],
    )
    def _sc_gather(table_hbm, idx_hbm, out_hbm, idx_v, rows_v, sem):
        wid = lax.axis_index("s") * _NC + lax.axis_index("c")
        base = wid * _PER_W
        pltpu.sync_copy(idx_hbm.at[pl.ds(base, _PER_W)], idx_v)
        for c in range(_PER_W // _CHUNK):
            pltpu.async_copy(
                table_hbm.at[idx_v.at[pl.ds(c * _CHUNK, _CHUNK)]], rows_v, sem
            ).wait()
            pltpu.sync_copy(rows_v,
                            out_hbm.at[pl.ds(base + c * _CHUNK, _CHUNK)])

    return _sc_gather


def _mm(a, b):
    return lax.dot_general(a, b, (((1,), (0,)), ((), ())),
                           preferred_element_type=jnp.float32)


def _mmT(a, b):
    return lax.dot_general(a, b, (((0,), (0,)), ((), ())),
                           preferred_element_type=jnp.float32)


def _gru_step(A, s, W_ref, U_ref, b_ref, mask, transpose):
    sb = s.astype(jnp.bfloat16)
    if transpose:
        m = _mmT(A, sb)
    else:
        m = _mm(A, sb)
    mb = m.astype(jnp.bfloat16)
    z = jax.nn.sigmoid(_mm(mb, W_ref[0]) + _mm(sb, U_ref[0]) + b_ref[0])
    r = jax.nn.sigmoid(_mm(mb, W_ref[1]) + _mm(sb, U_ref[1]) + b_ref[1])
    n = jnp.tanh(_mm(mb, W_ref[2]) + r * _mm(sb, U_ref[2]) + b_ref[2])
    return ((1.0 - z) * s + z * n) * mask


def _tc_body(len_ref, twf_ref, twb_ref, g_ref, tok_ref, deps_ref, et_ref,
             cm_ref, Wf_ref, Uf_ref, bf_ref, Wb_ref, Ub_ref, bb_ref,
             Wo1_ref, Wo2_ref, bo_ref, out_ref, hid_ref):
    b = pl.program_id(0)
    g3 = g_ref[...]
    tok = tok_ref[0]
    e = jnp.zeros((L, H), jnp.float32)
    for s in range(S):
        pad = (tok[:, s:s + 1] != 0).astype(jnp.float32)
        e = e + g3[s] * pad
    lenb = len_ref[b]
    mask = (lax.broadcasted_iota(jnp.int32, (L, 1), 0) < lenb
            ).astype(jnp.float32)
    sf = e
    sbk = e
    for r in range(R):
        base = deps_ref[0, r] * cm_ref[0, r] * jnp.float32(1.0 / L)
        et = et_ref[0, r]
        accf = jnp.take_along_axis(
            jnp.broadcast_to(twf_ref[...], (L, NT)), et, axis=1)
        accb = jnp.take_along_axis(
            jnp.broadcast_to(twb_ref[...], (L, NT)), et, axis=1)
        Af = (base * accf).astype(jnp.bfloat16)
        Ab = (base * accb).astype(jnp.bfloat16)
        sf = _gru_step(Af, sf, Wf_ref, Uf_ref, bf_ref, mask, False)
        sbk = _gru_step(Ab, sbk, Wb_ref, Ub_ref, bb_ref, mask, True)
    sfb = sf.astype(jnp.bfloat16)
    sbb = sbk.astype(jnp.bfloat16)
    out_ref[0] = _mm(sfb, Wo1_ref[...]) + _mm(sbb, Wo2_ref[...]) + bo_ref[0]
    inv = jnp.float32(1.0) / jnp.maximum(lenb, 1).astype(jnp.float32)
    hf = (sf.sum(axis=0, keepdims=True) * inv).astype(jnp.bfloat16)
    hb = (sbk.sum(axis=0, keepdims=True) * inv).astype(jnp.bfloat16)
    hid_ref[0] = _mm(hf, Wo1_ref[...]) + _mm(hb, Wo2_ref[...]) + bo_ref[0]


def _tc_call(lengths, tw_f, tw_b, g4, tok4, deps, edge_types, cell_mask,
             Wfb, Ufb, bf, Wbb, Ubb, bb, Wo1, Wo2, bo):
    smem = pl.BlockSpec(memory_space=pltpu.SMEM)
    full3 = lambda shape: pl.BlockSpec(shape, lambda b: (0, 0, 0))
    full2 = lambda shape: pl.BlockSpec(shape, lambda b: (0, 0))
    return pl.pallas_call(
        _tc_body,
        grid=(B,),
        in_specs=[
            smem,
            pl.BlockSpec((NT,), lambda b: (0,)),
            pl.BlockSpec((NT,), lambda b: (0,)),
            pl.BlockSpec((S, L, H), lambda b: (0, b, 0)),
            pl.BlockSpec((1, L, S), lambda b: (b, 0, 0)),
            pl.BlockSpec((1, R, L, L), lambda b: (b, 0, 0, 0)),
            pl.BlockSpec((1, R, L, L), lambda b: (b, 0, 0, 0)),
            pl.BlockSpec((1, R, L, L), lambda b: (b, 0, 0, 0)),
            full3((3, H, H)),
            full3((3, H, H)),
            full2((3, H)),
            full3((3, H, H)),
            full3((3, H, H)),
            full2((3, H)),
            full2((H, H)),
            full2((H, H)),
            full2((1, H)),
        ],
        out_specs=[
            pl.BlockSpec((1, L, H), lambda b: (b, 0, 0)),
            pl.BlockSpec((1, 1, H), lambda b: (b, 0, 0)),
        ],
        out_shape=[
            jax.ShapeDtypeStruct((B, L, H), jnp.float32),
            jax.ShapeDtypeStruct((B, 1, H), jnp.float32),
        ],
    )(lengths, tw_f, tw_b, g4, tok4, deps, edge_types, cell_mask,
      Wfb, Ufb, bf, Wbb, Ubb, bb, Wo1, Wo2, bo)


def kernel(input_seqs, input_lengths, deps, edge_types, cell_mask, emb,
           type_w_f, type_w_b, Wf, Uf, bf, Wb, Ub, bb, W_out, b_out):
    flat_idx = input_seqs.reshape(B * L, S).T.reshape(_TOTAL)
    gathered = _sc_gather_fn()(emb, flat_idx).reshape(S, B * L, H)
    bf16 = jnp.bfloat16
    outputs, hidden3 = _tc_call(
        input_lengths, type_w_f, type_w_b, gathered, input_seqs, deps,
        edge_types,
        cell_mask, Wf.astype(bf16), Uf.astype(bf16), bf,
        Wb.astype(bf16), Ub.astype(bf16), bb,
        W_out[:H].astype(bf16), W_out[H:].astype(bf16), b_out.reshape(1, H))
    return outputs, hidden3.reshape(B, H)
